# Initial kernel scaffold; baseline (speedup 1.0000x reference)
import jax
import jax.numpy as jnp
from jax.experimental import pallas as pl


def kernel(class_indexes, value_indexes, base_predictor, W_present, W_value, W_query, position_embed):
    raise NotImplementedError("write your pallas kernel here")


# speedup vs baseline: 2.0101x; 2.0101x over previous
import functools

import jax
import jax.numpy as jnp
from jax import lax
from jax.experimental import pallas as pl
from jax.experimental.pallas import tpu as pltpu
from jax.experimental.pallas import tpu_sc as plsc

NUM_CLASSES = 1000
D = 128
K = 20
B = 1024
S = 4
N = B * S
LEVELS = 128
EMBED_SCALE = float(D) ** 0.5

NC = 2
NS = 16
NW = NC * NS
SEQ_PER_W = N // NW
CH = 4
NCHUNK = SEQ_PER_W // CH
ROWS = CH * K
NV = D // 16


def _tec_body(ci_hbm, vals_hbm, bp_hbm, wp_hbm, wv_hbm, wpq_hbm, pos_hbm,
              cp_hbm, vp_hbm,
              pos_v, bp_row, ci_ch, vals_ch, p_rows, v_rows, pq_rows,
              cp_out, vp_out, semg, semo):
    wid = lax.axis_index("s") * NC + lax.axis_index("c")

    pltpu.sync_copy(pos_hbm, pos_v)

    def chunk_body(c, _):
        goff = (wid * SEQ_PER_W + c * CH) * K
        b_row = wid * NCHUNK + c

        pltpu.sync_copy(ci_hbm.at[pl.ds(goff, ROWS)], ci_ch)
        pltpu.sync_copy(vals_hbm.at[pl.ds(goff, ROWS)], vals_ch)
        pltpu.sync_copy(bp_hbm.at[pl.ds(b_row, 1)], bp_row)

        cp1 = pltpu.async_copy(wp_hbm.at[ci_ch], p_rows, semg)
        cp2 = pltpu.async_copy(wv_hbm.at[ci_ch], v_rows, semg)
        cp3 = pltpu.async_copy(wpq_hbm.at[ci_ch], pq_rows, semg)
        cp1.wait()
        cp2.wait()
        cp3.wait()

        for s_local in range(CH):
            acc = [bp_row[0, pl.ds(dv * 16, 16)] for dv in range(NV)]
            for k in range(K):
                r = s_local * K + k
                val = vals_ch[r, :]
                for dv in range(NV):
                    sl = pl.ds(dv * 16, 16)
                    p = p_rows[r, sl]
                    v = v_rows[r, sl]
                    pq = pq_rows[r, sl]
                    cp = acc[dv] + pos_v[k, sl]
                    cp_out[r, sl] = cp
                    vp_out[r, sl] = cp + pq
                    acc[dv] = acc[dv] + p + val * v

        o1 = pltpu.async_copy(cp_out, cp_hbm.at[pl.ds(goff, ROWS)], semo)
        o2 = pltpu.async_copy(vp_out, vp_hbm.at[pl.ds(goff, ROWS)], semo)
        o1.wait()
        o2.wait()
        return ()

    lax.fori_loop(0, NCHUNK, chunk_body, (), unroll=False)


@jax.jit
def _predictor_sc(ci_flat, vals_flat, base_predictor, wp_s, wv_s, wpq_s, pos_s):
    mesh = plsc.VectorSubcoreMesh(core_axis_name="c", subcore_axis_name="s")
    f = pl.kernel(
        _tec_body,
        out_type=(
            jax.ShapeDtypeStruct((N * K, D), jnp.float32),
            jax.ShapeDtypeStruct((N * K, D), jnp.float32),
        ),
        mesh=mesh,
        scratch_types=[
            pltpu.VMEM((K, D), jnp.float32),
            pltpu.VMEM((1, D), jnp.float32),
            pltpu.VMEM((ROWS,), jnp.int32),
            pltpu.VMEM((ROWS, 16), jnp.float32),
            pltpu.VMEM((ROWS, D), jnp.float32),
            pltpu.VMEM((ROWS, D), jnp.float32),
            pltpu.VMEM((ROWS, D), jnp.float32),
            pltpu.VMEM((ROWS, D), jnp.float32),
            pltpu.VMEM((ROWS, D), jnp.float32),
            pltpu.SemaphoreType.DMA,
            pltpu.SemaphoreType.DMA,
        ],
    )
    return f(ci_flat, vals_flat, base_predictor, wp_s, wv_s, wpq_s, pos_s)


def kernel(class_indexes, value_indexes, base_predictor, W_present, W_value,
           W_query, position_embed):
    ci_flat = class_indexes.reshape(N * K)
    vals_flat = jnp.broadcast_to(
        (value_indexes.astype(jnp.float32)
         * (1.0 / (LEVELS - 1))).reshape(N * K, 1), (N * K, 16))
    wp_s = W_present * EMBED_SCALE
    wv_s = W_value * EMBED_SCALE
    wpq_s = (W_present + W_query) * EMBED_SCALE
    pos_s = position_embed * EMBED_SCALE
    cp, vp = _predictor_sc(ci_flat, vals_flat, base_predictor,
                           wp_s, wv_s, wpq_s, pos_s)
    return (cp.reshape(B, S, K, D), vp.reshape(B, S, K, D))


# speedup vs baseline: 2.8151x; 1.4005x over previous
import jax
import jax.numpy as jnp
from jax import lax
from jax.experimental import pallas as pl
from jax.experimental.pallas import tpu as pltpu
from jax.experimental.pallas import tpu_sc as plsc

NUM_CLASSES = 1000
D = 128
K = 20
B = 1024
S = 4
N = B * S
LEVELS = 128
EMBED_SCALE = float(D) ** 0.5

NC = 2
NS = 16
NW = NC * NS
SEQ_PER_W = N // NW
CH = 2
NCHUNK = SEQ_PER_W // CH
ROWS = CH * K
NV = D // 16


def _tec_body(ci_hbm, vals_hbm, bp_hbm, wp_hbm, wv_hbm, wpq_hbm, pos_hbm,
              cp_hbm, vp_hbm,
              pos_v, ci_all,
              bp0, bp1, vals0, vals1, p0, p1, v0, v1, pq0, pq1,
              cpo0, cpo1, vpo0, vpo1, semg0, semg1, semo0, semo1):
    wid = lax.axis_index("s") * NC + lax.axis_index("c")
    wbase = wid * SEQ_PER_W * K

    pltpu.sync_copy(pos_hbm, pos_v)
    pltpu.sync_copy(ci_hbm.at[pl.ds(wbase, SEQ_PER_W * K)], ci_all)

    bufs = (
        (bp0, vals0, p0, v0, pq0, cpo0, vpo0, semg0, semo0),
        (bp1, vals1, p1, v1, pq1, cpo1, vpo1, semg1, semo1),
    )

    def start(c, buf):
        bp, vals, p, v, pq, semg, _, _, _ = (
            bufs[buf][0], bufs[buf][1], bufs[buf][2], bufs[buf][3],
            bufs[buf][4], bufs[buf][7], None, None, None)
        goff = wbase + c * ROWS
        idx = ci_all.at[pl.ds(c * ROWS, ROWS)]
        pltpu.async_copy(
            bp_hbm.at[pl.ds(wid * SEQ_PER_W // S + c * CH // S, 1)], bp, semg)
        pltpu.async_copy(vals_hbm.at[pl.ds(goff, ROWS)], vals, semg)
        pltpu.async_copy(wp_hbm.at[idx], p, semg)
        pltpu.async_copy(wv_hbm.at[idx], v, semg)
        pltpu.async_copy(wpq_hbm.at[idx], pq, semg)

    def finish(c, buf, j):
        bp, vals, p, v, pq, cpo, vpo, semg, semo = bufs[buf]
        goff = wbase + c * ROWS
        pltpu.make_async_copy(bp_hbm.at[pl.ds(0, 1)], bp, semg).wait()
        pltpu.make_async_copy(vals_hbm.at[pl.ds(0, ROWS)], vals, semg).wait()
        pltpu.make_async_copy(wp_hbm.at[pl.ds(0, ROWS)], p, semg).wait()
        pltpu.make_async_copy(wv_hbm.at[pl.ds(0, ROWS)], v, semg).wait()
        pltpu.make_async_copy(wpq_hbm.at[pl.ds(0, ROWS)], pq, semg).wait()

        @pl.when(j >= 1)
        def _():
            pltpu.make_async_copy(cpo, cp_hbm.at[pl.ds(0, ROWS)], semo).wait()
            pltpu.make_async_copy(vpo, vp_hbm.at[pl.ds(0, ROWS)], semo).wait()

        for s_local in range(CH):
            acc = [bp[0, pl.ds(dv * 16, 16)] for dv in range(NV)]
            for k in range(K):
                r = s_local * K + k
                val = vals[r, :]
                for dv in range(NV):
                    sl = pl.ds(dv * 16, 16)
                    cp = acc[dv] + pos_v[k, sl]
                    cpo[r, sl] = cp
                    vpo[r, sl] = cp + pq[r, sl]
                    acc[dv] = acc[dv] + p[r, sl] + val * v[r, sl]

        pltpu.async_copy(cpo, cp_hbm.at[pl.ds(goff, ROWS)], semo)
        pltpu.async_copy(vpo, vp_hbm.at[pl.ds(goff, ROWS)], semo)

    start(0, 0)

    def body(j, _):
        c0 = 2 * j
        start(c0 + 1, 1)
        finish(c0, 0, j)

        @pl.when(j < NCHUNK // 2 - 1)
        def _():
            start(c0 + 2, 0)

        finish(c0 + 1, 1, j)
        return ()

    lax.fori_loop(0, NCHUNK // 2, body, (), unroll=False)

    pltpu.make_async_copy(cpo0, cp_hbm.at[pl.ds(0, ROWS)], semo0).wait()
    pltpu.make_async_copy(vpo0, vp_hbm.at[pl.ds(0, ROWS)], semo0).wait()
    pltpu.make_async_copy(cpo1, cp_hbm.at[pl.ds(0, ROWS)], semo1).wait()
    pltpu.make_async_copy(vpo1, vp_hbm.at[pl.ds(0, ROWS)], semo1).wait()


@jax.jit
def _predictor_sc(ci_flat, vals_flat, base_predictor, wp_s, wv_s, wpq_s, pos_s):
    mesh = plsc.VectorSubcoreMesh(core_axis_name="c", subcore_axis_name="s")
    f = pl.kernel(
        _tec_body,
        out_type=(
            jax.ShapeDtypeStruct((N * K, D), jnp.float32),
            jax.ShapeDtypeStruct((N * K, D), jnp.float32),
        ),
        mesh=mesh,
        scratch_types=[
            pltpu.VMEM((K, D), jnp.float32),
            pltpu.VMEM((SEQ_PER_W * K,), jnp.int32),
            pltpu.VMEM((1, D), jnp.float32),
            pltpu.VMEM((1, D), jnp.float32),
            pltpu.VMEM((ROWS, 16), jnp.float32),
            pltpu.VMEM((ROWS, 16), jnp.float32),
            pltpu.VMEM((ROWS, D), jnp.float32),
            pltpu.VMEM((ROWS, D), jnp.float32),
            pltpu.VMEM((ROWS, D), jnp.float32),
            pltpu.VMEM((ROWS, D), jnp.float32),
            pltpu.VMEM((ROWS, D), jnp.float32),
            pltpu.VMEM((ROWS, D), jnp.float32),
            pltpu.VMEM((ROWS, D), jnp.float32),
            pltpu.VMEM((ROWS, D), jnp.float32),
            pltpu.VMEM((ROWS, D), jnp.float32),
            pltpu.VMEM((ROWS, D), jnp.float32),
            pltpu.SemaphoreType.DMA,
            pltpu.SemaphoreType.DMA,
            pltpu.SemaphoreType.DMA,
            pltpu.SemaphoreType.DMA,
        ],
    )
    return f(ci_flat, vals_flat, base_predictor, wp_s, wv_s, wpq_s, pos_s)


def kernel(class_indexes, value_indexes, base_predictor, W_present, W_value,
           W_query, position_embed):
    ci_flat = class_indexes.reshape(N * K)
    vals_flat = jnp.broadcast_to(
        (value_indexes.astype(jnp.float32)
         * (1.0 / (LEVELS - 1))).reshape(N * K, 1), (N * K, 16))
    wp_s = W_present * EMBED_SCALE
    wv_s = W_value * EMBED_SCALE
    wpq_s = (W_present + W_query) * EMBED_SCALE
    pos_s = position_embed * EMBED_SCALE
    cp, vp = _predictor_sc(ci_flat, vals_flat, base_predictor,
                           wp_s, wv_s, wpq_s, pos_s)
    return (cp.reshape(B, S, K, D), vp.reshape(B, S, K, D))


# speedup vs baseline: 3.1336x; 1.1131x over previous
import jax
import jax.numpy as jnp
from jax import lax
from jax.experimental import pallas as pl
from jax.experimental.pallas import tpu as pltpu
from jax.experimental.pallas import tpu_sc as plsc

NUM_CLASSES = 1000
D = 128
K = 20
B = 1024
S = 4
N = B * S
LEVELS = 128
EMBED_SCALE = float(D) ** 0.5

NC = 2
NS = 16
NW = NC * NS
SEQ_PER_W = N // NW
CH = 2
NCHUNK = SEQ_PER_W // CH
ROWS = CH * K
NV = D // 16


def _tec_body(ci_hbm, vals_hbm, bp_hbm, wp_hbm, wv_hbm, wpq_hbm, pos_hbm,
              cp_hbm, vp_hbm,
              pos_v, ci_all,
              bp0, bp1, vals0, vals1, p0, p1, v0, v1, pq0, pq1,
              cpo0, cpo1, vpo0, vpo1, semg0, semg1, semo0, semo1):
    wid = lax.axis_index("s") * NC + lax.axis_index("c")
    wbase = wid * SEQ_PER_W * K

    pltpu.sync_copy(pos_hbm, pos_v)
    pltpu.sync_copy(ci_hbm.at[pl.ds(wbase, SEQ_PER_W * K)], ci_all)

    bufs = (
        (bp0, vals0, p0, v0, pq0, cpo0, vpo0, semg0, semo0),
        (bp1, vals1, p1, v1, pq1, cpo1, vpo1, semg1, semo1),
    )

    def start(c, buf):
        bp, vals, p, v, pq, semg, _, _, _ = (
            bufs[buf][0], bufs[buf][1], bufs[buf][2], bufs[buf][3],
            bufs[buf][4], bufs[buf][7], None, None, None)
        goff = wbase + c * ROWS
        idx = ci_all.at[pl.ds(c * ROWS, ROWS)]
        pltpu.async_copy(
            bp_hbm.at[pl.ds(wid * SEQ_PER_W // S + c * CH // S, 1)], bp, semg)
        pltpu.async_copy(vals_hbm.at[pl.ds(goff, ROWS)], vals, semg)
        pltpu.async_copy(wp_hbm.at[idx], p, semg)
        pltpu.async_copy(wv_hbm.at[idx], v, semg)
        pltpu.async_copy(wpq_hbm.at[idx], pq, semg)

    def finish(c, buf, j):
        bp, vals, p, v, pq, cpo, vpo, semg, semo = bufs[buf]
        b_row = wid * SEQ_PER_W // S + c * CH // S
        s0 = (c % (S // CH)) * CH
        pltpu.make_async_copy(bp_hbm.at[pl.ds(0, 1)], bp, semg).wait()
        pltpu.make_async_copy(vals_hbm.at[pl.ds(0, ROWS)], vals, semg).wait()
        pltpu.make_async_copy(wp_hbm.at[pl.ds(0, ROWS)], p, semg).wait()
        pltpu.make_async_copy(wv_hbm.at[pl.ds(0, ROWS)], v, semg).wait()
        pltpu.make_async_copy(wpq_hbm.at[pl.ds(0, ROWS)], pq, semg).wait()

        @pl.when(j >= 1)
        def _():
            pltpu.make_async_copy(cpo, cp_hbm.at[0, pl.ds(0, CH)], semo).wait()
            pltpu.make_async_copy(vpo, vp_hbm.at[0, pl.ds(0, CH)], semo).wait()

        for s_local in range(CH):
            for dv in range(NV):
                sl = pl.ds(dv * 16, 16)
                acc = bp[0, sl]
                for k in range(K):
                    r = s_local * K + k
                    cp = acc + pos_v[k, sl]
                    cpo[s_local, k, sl] = cp
                    vpo[s_local, k, sl] = cp + pq[r, sl]
                    acc = acc + p[r, sl] + vals[r, :] * v[r, sl]

        pltpu.async_copy(cpo, cp_hbm.at[b_row, pl.ds(s0, CH)], semo)
        pltpu.async_copy(vpo, vp_hbm.at[b_row, pl.ds(s0, CH)], semo)

    start(0, 0)

    def body(j, _):
        c0 = 2 * j
        start(c0 + 1, 1)
        finish(c0, 0, j)

        @pl.when(j < NCHUNK // 2 - 1)
        def _():
            start(c0 + 2, 0)

        finish(c0 + 1, 1, j)
        return ()

    lax.fori_loop(0, NCHUNK // 2, body, (), unroll=False)

    pltpu.make_async_copy(cpo0, cp_hbm.at[0, pl.ds(0, CH)], semo0).wait()
    pltpu.make_async_copy(vpo0, vp_hbm.at[0, pl.ds(0, CH)], semo0).wait()
    pltpu.make_async_copy(cpo1, cp_hbm.at[0, pl.ds(0, CH)], semo1).wait()
    pltpu.make_async_copy(vpo1, vp_hbm.at[0, pl.ds(0, CH)], semo1).wait()


@jax.jit
def _predictor_sc(ci_flat, vals_flat, base_predictor, wp_s, wv_s, wpq_s, pos_s):
    mesh = plsc.VectorSubcoreMesh(core_axis_name="c", subcore_axis_name="s")
    f = pl.kernel(
        _tec_body,
        out_type=(
            jax.ShapeDtypeStruct((B, S, K, D), jnp.float32),
            jax.ShapeDtypeStruct((B, S, K, D), jnp.float32),
        ),
        mesh=mesh,
        scratch_types=[
            pltpu.VMEM((K, D), jnp.float32),
            pltpu.VMEM((SEQ_PER_W * K,), jnp.int32),
            pltpu.VMEM((1, D), jnp.float32),
            pltpu.VMEM((1, D), jnp.float32),
            pltpu.VMEM((ROWS, 16), jnp.float32),
            pltpu.VMEM((ROWS, 16), jnp.float32),
            pltpu.VMEM((ROWS, D), jnp.float32),
            pltpu.VMEM((ROWS, D), jnp.float32),
            pltpu.VMEM((ROWS, D), jnp.float32),
            pltpu.VMEM((ROWS, D), jnp.float32),
            pltpu.VMEM((ROWS, D), jnp.float32),
            pltpu.VMEM((ROWS, D), jnp.float32),
            pltpu.VMEM((CH, K, D), jnp.float32),
            pltpu.VMEM((CH, K, D), jnp.float32),
            pltpu.VMEM((CH, K, D), jnp.float32),
            pltpu.VMEM((CH, K, D), jnp.float32),
            pltpu.SemaphoreType.DMA,
            pltpu.SemaphoreType.DMA,
            pltpu.SemaphoreType.DMA,
            pltpu.SemaphoreType.DMA,
        ],
    )
    return f(ci_flat, vals_flat, base_predictor, wp_s, wv_s, wpq_s, pos_s)


def kernel(class_indexes, value_indexes, base_predictor, W_present, W_value,
           W_query, position_embed):
    ci_flat = class_indexes.reshape(N * K)
    vals_flat = jnp.broadcast_to(
        (value_indexes.astype(jnp.float32)
         * (1.0 / (LEVELS - 1))).reshape(N * K, 1), (N * K, 16))
    wp_s = W_present * EMBED_SCALE
    wv_s = W_value * EMBED_SCALE
    wpq_s = (W_present + W_query) * EMBED_SCALE
    pos_s = position_embed * EMBED_SCALE
    cp, vp = _predictor_sc(ci_flat, vals_flat, base_predictor,
                           wp_s, wv_s, wpq_s, pos_s)
    return (cp, vp)


# speedup vs baseline: 3.7881x; 1.2089x over previous
import jax
import jax.numpy as jnp
from jax import lax
from jax.experimental import pallas as pl
from jax.experimental.pallas import tpu as pltpu
from jax.experimental.pallas import tpu_sc as plsc

NUM_CLASSES = 1000
D = 128
K = 20
B = 1024
S = 4
N = B * S
LEVELS = 128
EMBED_SCALE = float(D) ** 0.5

NC = 2
NS = 16
NW = NC * NS
SEQ_PER_W = N // NW
CH = 2
NCHUNK = SEQ_PER_W // CH
ROWS = CH * K
NV = D // 16
NBUF = 4
VPAD = 48


def _tec_body(ci_hbm, vals_hbm, bp_hbm, wp_hbm, wv_hbm, wpq_hbm, pos_hbm,
              cp_hbm, vp_hbm, pos_v, ci_all, valx, *ring):
    bps = ring[0:NBUF]
    valss = ring[NBUF:2 * NBUF]
    ps = ring[2 * NBUF:3 * NBUF]
    vs = ring[3 * NBUF:4 * NBUF]
    pqs = ring[4 * NBUF:5 * NBUF]
    cpos = ring[5 * NBUF:6 * NBUF]
    vpos = ring[6 * NBUF:7 * NBUF]
    semgs = ring[7 * NBUF:8 * NBUF]
    semos = ring[8 * NBUF:9 * NBUF]

    wid = lax.axis_index("s") * NC + lax.axis_index("c")
    wbase = wid * SEQ_PER_W * K

    pltpu.sync_copy(pos_hbm, pos_v)
    pltpu.sync_copy(ci_hbm.at[pl.ds(wbase, SEQ_PER_W * K)], ci_all)

    def start(c, b):
        goff = wbase + c * ROWS
        idx = ci_all.at[pl.ds(c * ROWS, ROWS)]
        pltpu.async_copy(
            bp_hbm.at[pl.ds(wid * SEQ_PER_W // S + c * CH // S, 1)],
            bps[b], semgs[b])
        pltpu.async_copy(vals_hbm.at[pl.ds(goff, ROWS)],
                         valss[b].at[pl.ds(0, ROWS)], semgs[b])
        pltpu.async_copy(wp_hbm.at[idx], ps[b], semgs[b])
        pltpu.async_copy(wv_hbm.at[idx], vs[b], semgs[b])
        pltpu.async_copy(wpq_hbm.at[idx], pqs[b], semgs[b])

    def finish(c, b, j):
        bp, vals, p, v, pq = bps[b], valss[b], ps[b], vs[b], pqs[b]
        cpo, vpo, semg, semo = cpos[b], vpos[b], semgs[b], semos[b]
        b_row = wid * SEQ_PER_W // S + c * CH // S
        s0 = (c % (S // CH)) * CH
        pltpu.make_async_copy(bp_hbm.at[pl.ds(0, 1)], bp, semg).wait()
        pltpu.make_async_copy(vals_hbm.at[pl.ds(0, ROWS)],
                              vals.at[pl.ds(0, ROWS)], semg).wait()
        pltpu.make_async_copy(wp_hbm.at[pl.ds(0, ROWS)], p, semg).wait()
        pltpu.make_async_copy(wv_hbm.at[pl.ds(0, ROWS)], v, semg).wait()
        pltpu.make_async_copy(wpq_hbm.at[pl.ds(0, ROWS)], pq, semg).wait()

        @pl.when(j >= 1)
        def _():
            pltpu.make_async_copy(cpo, cp_hbm.at[0, pl.ds(0, CH)], semo).wait()
            pltpu.make_async_copy(vpo, vp_hbm.at[0, pl.ds(0, CH)], semo).wait()

        for r in range(ROWS):
            g = vals[pl.ds((r // 16) * 16, 16)]
            valx[r, :] = lax.broadcast_in_dim(
                lax.slice_in_dim(g, r % 16, r % 16 + 1), (16,), (0,))

        for s_local in range(CH):
            def dv_body(dv, _):
                sl = pl.ds(dv * 16, 16)
                acc = bp[0, sl]
                for k in range(K):
                    r = s_local * K + k
                    cp = acc + pos_v[k, sl]
                    cpo[s_local, k, sl] = cp
                    vpo[s_local, k, sl] = cp + pq[r, sl]
                    acc = acc + p[r, sl] + valx[r, :] * v[r, sl]
                return ()

            lax.fori_loop(0, NV, dv_body, (), unroll=False)

        pltpu.async_copy(cpo, cp_hbm.at[b_row, pl.ds(s0, CH)], semo)
        pltpu.async_copy(vpo, vp_hbm.at[b_row, pl.ds(s0, CH)], semo)

    for b in range(NBUF):
        start(b, b)

    def body(j, _):
        for b in range(NBUF):
            c = j * NBUF + b
            finish(c, b, j)

            @pl.when(c + NBUF < NCHUNK)
            def _():
                start(c + NBUF, b)
        return ()

    lax.fori_loop(0, NCHUNK // NBUF, body, (), unroll=False)

    for b in range(NBUF):
        pltpu.make_async_copy(cpos[b], cp_hbm.at[0, pl.ds(0, CH)],
                              semos[b]).wait()
        pltpu.make_async_copy(vpos[b], vp_hbm.at[0, pl.ds(0, CH)],
                              semos[b]).wait()


@jax.jit
def _predictor_sc(ci_flat, vals_flat, base_predictor, wp_s, wv_s, wpq_s, pos_s):
    mesh = plsc.VectorSubcoreMesh(core_axis_name="c", subcore_axis_name="s")
    scratch = [
        pltpu.VMEM((K, D), jnp.float32),
        pltpu.VMEM((SEQ_PER_W * K,), jnp.int32),
        pltpu.VMEM((ROWS, 16), jnp.float32),
    ]
    scratch += [pltpu.VMEM((1, D), jnp.float32) for _ in range(NBUF)]
    scratch += [pltpu.VMEM((VPAD,), jnp.float32) for _ in range(NBUF)]
    scratch += [pltpu.VMEM((ROWS, D), jnp.float32) for _ in range(NBUF)]
    scratch += [pltpu.VMEM((ROWS, D), jnp.float32) for _ in range(NBUF)]
    scratch += [pltpu.VMEM((ROWS, D), jnp.float32) for _ in range(NBUF)]
    scratch += [pltpu.VMEM((CH, K, D), jnp.float32) for _ in range(NBUF)]
    scratch += [pltpu.VMEM((CH, K, D), jnp.float32) for _ in range(NBUF)]
    scratch += [pltpu.SemaphoreType.DMA for _ in range(NBUF)]
    scratch += [pltpu.SemaphoreType.DMA for _ in range(NBUF)]
    f = pl.kernel(
        _tec_body,
        out_type=(
            jax.ShapeDtypeStruct((B, S, K, D), jnp.float32),
            jax.ShapeDtypeStruct((B, S, K, D), jnp.float32),
        ),
        mesh=mesh,
        scratch_types=scratch,
    )
    return f(ci_flat, vals_flat, base_predictor, wp_s, wv_s, wpq_s, pos_s)


def kernel(class_indexes, value_indexes, base_predictor, W_present, W_value,
           W_query, position_embed):
    ci_flat = class_indexes.reshape(N * K)
    vals_flat = (value_indexes.astype(jnp.float32)
                 * (1.0 / (LEVELS - 1))).reshape(N * K)
    wp_s = W_present * EMBED_SCALE
    wv_s = W_value * EMBED_SCALE
    wpq_s = (W_present + W_query) * EMBED_SCALE
    pos_s = position_embed * EMBED_SCALE
    return _predictor_sc(ci_flat, vals_flat, base_predictor,
                         wp_s, wv_s, wpq_s, pos_s)


# speedup vs baseline: 3.8534x; 1.0172x over previous
import jax
import jax.numpy as jnp
from jax import lax
from jax.experimental import pallas as pl
from jax.experimental.pallas import tpu as pltpu
from jax.experimental.pallas import tpu_sc as plsc

NUM_CLASSES = 1000
D = 128
K = 20
B = 1024
S = 4
N = B * S
LEVELS = 128
EMBED_SCALE = float(D) ** 0.5

NC = 2
NS = 16
NW = NC * NS
SEQ_PER_W = N // NW
CH = 2
NCHUNK = SEQ_PER_W // CH
ROWS = CH * K
NV = D // 16
NBUF = 4
VPAD = 48


def _tec_body(ci_hbm, vals_hbm, bp_hbm, wcat_hbm, pos_hbm,
              cp_hbm, vp_hbm, pos_v, ci_all, valx, *ring):
    bps = ring[0:NBUF]
    valss = ring[NBUF:2 * NBUF]
    cats = ring[2 * NBUF:3 * NBUF]
    cpos = ring[3 * NBUF:4 * NBUF]
    vpos = ring[4 * NBUF:5 * NBUF]
    semgs = ring[5 * NBUF:6 * NBUF]
    semos = ring[6 * NBUF:7 * NBUF]

    wid = lax.axis_index("s") * NC + lax.axis_index("c")
    wbase = wid * SEQ_PER_W * K

    pltpu.sync_copy(pos_hbm, pos_v)
    pltpu.sync_copy(ci_hbm.at[pl.ds(wbase, SEQ_PER_W * K)], ci_all)

    def start(c, b):
        goff = wbase + c * ROWS
        idx = ci_all.at[pl.ds(c * ROWS, ROWS)]
        pltpu.async_copy(
            bp_hbm.at[pl.ds(wid * SEQ_PER_W // S + c * CH // S, 1)],
            bps[b], semgs[b])
        pltpu.async_copy(vals_hbm.at[pl.ds(goff, ROWS)],
                         valss[b].at[pl.ds(0, ROWS)], semgs[b])
        pltpu.async_copy(wcat_hbm.at[idx], cats[b], semgs[b])

    def finish(c, b, j):
        bp, vals, cat = bps[b], valss[b], cats[b]
        cpo, vpo, semg, semo = cpos[b], vpos[b], semgs[b], semos[b]
        b_row = wid * SEQ_PER_W // S + c * CH // S
        s0 = (c % (S // CH)) * CH
        pltpu.make_async_copy(bp_hbm.at[pl.ds(0, 1)], bp, semg).wait()
        pltpu.make_async_copy(vals_hbm.at[pl.ds(0, ROWS)],
                              vals.at[pl.ds(0, ROWS)], semg).wait()
        pltpu.make_async_copy(wcat_hbm.at[pl.ds(0, ROWS)], cat, semg).wait()

        @pl.when(j >= 1)
        def _():
            pltpu.make_async_copy(cpo, cp_hbm.at[0, pl.ds(0, CH)], semo).wait()
            pltpu.make_async_copy(vpo, vp_hbm.at[0, pl.ds(0, CH)], semo).wait()

        for r in range(ROWS):
            g = vals[pl.ds((r // 16) * 16, 16)]
            valx[r, :] = lax.broadcast_in_dim(
                lax.slice_in_dim(g, r % 16, r % 16 + 1), (16,), (0,))

        for s_local in range(CH):
            def dv_body(dv, _):
                sl = pl.ds(dv * 16, 16)
                acc = bp[0, sl]
                for k in range(K):
                    r = s_local * K + k
                    cp = acc + pos_v[k, sl]
                    cpo[s_local, k, sl] = cp
                    vpo[s_local, k, sl] = cp + cat[r, pl.ds(dv * 16 + 2 * D, 16)]
                    acc = (acc + cat[r, sl]
                           + valx[r, :] * cat[r, pl.ds(dv * 16 + D, 16)])
                return ()

            lax.fori_loop(0, NV, dv_body, (), unroll=False)

        pltpu.async_copy(cpo, cp_hbm.at[b_row, pl.ds(s0, CH)], semo)
        pltpu.async_copy(vpo, vp_hbm.at[b_row, pl.ds(s0, CH)], semo)

    for b in range(NBUF):
        start(b, b)

    def body(j, _):
        for b in range(NBUF):
            c = j * NBUF + b
            finish(c, b, j)

            @pl.when(c + NBUF < NCHUNK)
            def _():
                start(c + NBUF, b)
        return ()

    lax.fori_loop(0, NCHUNK // NBUF, body, (), unroll=False)

    for b in range(NBUF):
        pltpu.make_async_copy(cpos[b], cp_hbm.at[0, pl.ds(0, CH)],
                              semos[b]).wait()
        pltpu.make_async_copy(vpos[b], vp_hbm.at[0, pl.ds(0, CH)],
                              semos[b]).wait()


@jax.jit
def _predictor_sc(ci_flat, vals_flat, base_predictor, wcat, pos_s):
    mesh = plsc.VectorSubcoreMesh(core_axis_name="c", subcore_axis_name="s")
    scratch = [
        pltpu.VMEM((K, D), jnp.float32),
        pltpu.VMEM((SEQ_PER_W * K,), jnp.int32),
        pltpu.VMEM((ROWS, 16), jnp.float32),
    ]
    scratch += [pltpu.VMEM((1, D), jnp.float32) for _ in range(NBUF)]
    scratch += [pltpu.VMEM((VPAD,), jnp.float32) for _ in range(NBUF)]
    scratch += [pltpu.VMEM((ROWS, 3 * D), jnp.float32) for _ in range(NBUF)]
    scratch += [pltpu.VMEM((CH, K, D), jnp.float32) for _ in range(NBUF)]
    scratch += [pltpu.VMEM((CH, K, D), jnp.float32) for _ in range(NBUF)]
    scratch += [pltpu.SemaphoreType.DMA for _ in range(NBUF)]
    scratch += [pltpu.SemaphoreType.DMA for _ in range(NBUF)]
    f = pl.kernel(
        _tec_body,
        out_type=(
            jax.ShapeDtypeStruct((B, S, K, D), jnp.float32),
            jax.ShapeDtypeStruct((B, S, K, D), jnp.float32),
        ),
        mesh=mesh,
        scratch_types=scratch,
    )
    return f(ci_flat, vals_flat, base_predictor, wcat, pos_s)


def kernel(class_indexes, value_indexes, base_predictor, W_present, W_value,
           W_query, position_embed):
    ci_flat = class_indexes.reshape(N * K)
    vals_flat = (value_indexes.astype(jnp.float32)
                 * (1.0 / (LEVELS - 1))).reshape(N * K)
    wcat = jnp.concatenate(
        [W_present, W_value, W_present + W_query], axis=1) * EMBED_SCALE
    pos_s = position_embed * EMBED_SCALE
    return _predictor_sc(ci_flat, vals_flat, base_predictor, wcat, pos_s)
